# baseline (speedup 1.0000x reference)
import functools

import jax
import jax.numpy as jnp
from jax import lax
from jax.experimental import pallas as pl
from jax.experimental.pallas import tpu as pltpu
from jax.experimental.pallas import tpu_sc as plsc

D_MODEL = 1024
NUM_ROWS = 4
NUM_CORES = 2
NUM_SUBCORES = 16
NUM_WORKERS = NUM_CORES * NUM_SUBCORES

GROUP = 16
NSEM = 4


def _make_sc_lookup(batch: int):
    assert batch % (8 * NUM_WORKERS) == 0
    b_per_w = batch // NUM_WORKERS
    assert b_per_w % (GROUP * NSEM) == 0
    n_groups = b_per_w // GROUP

    mesh = plsc.VectorSubcoreMesh(
        core_axis_name="c", subcore_axis_name="s", num_cores=NUM_CORES
    )

    @functools.partial(
        pl.kernel,
        mesh=mesh,
        compiler_params=pltpu.CompilerParams(needs_layout_passes=False),
        out_type=jax.ShapeDtypeStruct((batch, D_MODEL), jnp.float32),
        scratch_types=[
            pltpu.VMEM((b_per_w,), jnp.int32),
            pltpu.VMEM((NUM_ROWS, D_MODEL), jnp.float32),
            pltpu.SemaphoreType.DMA((NSEM,)),
        ],
    )
    def lookup(ids_hbm, table_hbm, out_hbm, idx_v, table_v, sems):
        wid = lax.axis_index("s") * NUM_CORES + lax.axis_index("c")
        base = wid * b_per_w
        pltpu.sync_copy(table_hbm, table_v)
        pltpu.sync_copy(ids_hbm.at[pl.ds(base, b_per_w)], idx_v)

        def fire(g, s):
            idvec = idx_v[pl.ds(g * GROUP, GROUP)]
            for j in range(GROUP):
                row = g * GROUP + j
                rid = idvec[j]
                pltpu.make_async_copy(
                    table_v.at[rid],
                    out_hbm.at[base + row],
                    sems.at[s],
                ).start()

        def drain(s):
            for j in range(GROUP):
                pltpu.make_async_copy(
                    table_v.at[0],
                    out_hbm.at[0],
                    sems.at[s],
                ).wait()

        for s in range(NSEM):
            fire(s, s)

        def outer(o, _):
            for s in range(NSEM):
                g = o * NSEM + s
                drain(s)

                @pl.when(g + NSEM < n_groups)
                def _():
                    fire(g + NSEM, s)

            return ()

        lax.fori_loop(0, n_groups // NSEM, outer, (), unroll=False)

    return lookup


def kernel(segment_ids, table):
    b, s = segment_ids.shape
    ids_flat = segment_ids.reshape(b * s).astype(jnp.int32)
    out_flat = _make_sc_lookup(b * s)(ids_flat, table)
    return out_flat.reshape(b, s, D_MODEL)


# speedup vs baseline: 1.0620x; 1.0620x over previous
import functools

import jax
import jax.numpy as jnp
from jax import lax
from jax.experimental import pallas as pl
from jax.experimental.pallas import tpu as pltpu
from jax.experimental.pallas import tpu_sc as plsc

D_MODEL = 1024
NUM_ROWS = 4
NUM_CORES = 2
NUM_SUBCORES = 16
NUM_WORKERS = NUM_CORES * NUM_SUBCORES

GROUP = 16
NSEM = 2


def _make_sc_lookup(batch: int):
    assert batch % (8 * NUM_WORKERS) == 0
    b_per_w = batch // NUM_WORKERS
    assert b_per_w % (GROUP * NSEM) == 0
    n_groups = b_per_w // GROUP

    mesh = plsc.VectorSubcoreMesh(
        core_axis_name="c", subcore_axis_name="s", num_cores=NUM_CORES
    )

    @functools.partial(
        pl.kernel,
        mesh=mesh,
        compiler_params=pltpu.CompilerParams(needs_layout_passes=False),
        out_type=jax.ShapeDtypeStruct((batch, D_MODEL), jnp.float32),
        scratch_types=[
            pltpu.VMEM((b_per_w,), jnp.int32),
            pltpu.VMEM((GROUP, D_MODEL), jnp.float32),
            pltpu.SemaphoreType.DMA((NSEM,)),
        ],
    )
    def lookup(ids_hbm, table_hbm, out_hbm, idx_v, table_v, sems):
        wid = lax.axis_index("s") * NUM_CORES + lax.axis_index("c")
        base = wid * b_per_w
        pltpu.sync_copy(table_hbm, table_v.at[pl.ds(0, NUM_ROWS)])
        pltpu.sync_copy(ids_hbm.at[pl.ds(base, b_per_w)], idx_v)

        def fire(g, s):
            idvec = idx_v[pl.ds(g * GROUP, GROUP)]
            for j in range(GROUP):
                row = g * GROUP + j
                rid = idvec[j]
                pltpu.make_async_copy(
                    table_v.at[rid],
                    out_hbm.at[base + row],
                    sems.at[s],
                ).start()

        def drain(g, s):
            pltpu.make_async_copy(
                table_v,
                out_hbm.at[pl.ds(base + g * GROUP, GROUP)],
                sems.at[s],
            ).wait()

        fire(0, 0)

        def outer(o, _):
            for s in range(NSEM):
                g = o * NSEM + s

                @pl.when(g + 1 < n_groups)
                def _():
                    fire(g + 1, 1 - s)

                drain(g, s)
            return ()

        lax.fori_loop(0, n_groups // NSEM, outer, (), unroll=False)

    return lookup


def kernel(segment_ids, table):
    b, s = segment_ids.shape
    ids_flat = segment_ids.reshape(b * s).astype(jnp.int32)
    out_flat = _make_sc_lookup(b * s)(ids_flat, table)
    return out_flat.reshape(b, s, D_MODEL)
